# baseline (speedup 1.0000x reference)
import jax
import jax.numpy as jnp
from jax.experimental import pallas as pl
from jax.experimental.pallas import tpu as pltpu

_EPS = 1e-5


def _build_band(wt_ref, wb_scratch, W, C):
    WC = W * C
    u = jax.lax.broadcasted_iota(jnp.int32, (WC, WC), 0) // C
    v = jax.lax.broadcasted_iota(jnp.int32, (WC, WC), 1) // C
    d = v - u
    for dy in range(3):
        sec = jnp.zeros((WC, WC), jnp.float32)
        for dx in range(3):
            blk = jnp.broadcast_to(wt_ref[3 * dy + dx][None], (W, C, WC))
            sec = jnp.where(d == 1 - dx, blk.reshape(WC, WC), sec)
        wb_scratch[dy * WC:(dy + 1) * WC, :] = sec.astype(jnp.bfloat16)


def _tile_w(w, W):
    C = w.shape[-1]
    return jnp.tile(w.reshape(9, C, C), (1, 1, W))


def _conv_rows(xs, wb_scratch, H):
    M, WC = xs.shape
    row = jax.lax.broadcasted_iota(jnp.int32, (M, 1), 0)
    x_up = jnp.where(row % H != 0, pltpu.roll(xs, 1, axis=0), 0.0)
    x_dn = jnp.where(row % H != (H - 1), pltpu.roll(xs, M - 1, axis=0), 0.0)
    xcat = jnp.concatenate([x_up, xs, x_dn], axis=1).astype(jnp.bfloat16)
    return jnp.dot(xcat, wb_scratch[...], preferred_element_type=jnp.float32)


def _stats(acc, s_ref, s2_ref):
    s_ref[...] = jnp.sum(acc, axis=0, keepdims=True)[None]
    s2_ref[...] = jnp.sum(acc * acc, axis=0, keepdims=True)[None]


def _conv1_kernel(x_ref, wt_ref, o_ref, s_ref, s2_ref, wb_scratch):
    B, H, WC = x_ref.shape
    C = wt_ref.shape[1]

    @pl.when(pl.program_id(0) == 0)
    def _():
        _build_band(wt_ref, wb_scratch, WC // C, C)

    acc = _conv_rows(x_ref[...].reshape(B * H, WC), wb_scratch, H)
    o_ref[...] = acc.reshape(B, H, WC).astype(o_ref.dtype)
    _stats(acc, s_ref, s2_ref)


def _bn_relu_conv2_stats_kernel(c_ref, scale_ref, shift_ref, wt_ref,
                                s_ref, s2_ref, wb_scratch):
    B, H, WC = c_ref.shape
    C = wt_ref.shape[1]

    @pl.when(pl.program_id(0) == 0)
    def _():
        _build_band(wt_ref, wb_scratch, WC // C, C)

    c = c_ref[...].astype(jnp.float32)
    h = jnp.maximum(c * scale_ref[...] + shift_ref[...], 0.0)
    acc = _conv_rows(h.reshape(B * H, WC), wb_scratch, H)
    _stats(acc, s_ref, s2_ref)


def _conv2_finish_kernel(c_ref, x_ref, scale1_ref, shift1_ref,
                         scale2_ref, shift2_ref, wt_ref, o_ref, wb_scratch):
    B, H, WC = c_ref.shape
    C = wt_ref.shape[1]

    @pl.when(pl.program_id(0) == 0)
    def _():
        _build_band(wt_ref, wb_scratch, WC // C, C)

    c = c_ref[...].astype(jnp.float32)
    h = jnp.maximum(c * scale1_ref[...] + shift1_ref[...], 0.0)
    acc = _conv_rows(h.reshape(B * H, WC), wb_scratch, H).reshape(B, H, WC)
    y = acc * scale2_ref[...] + shift2_ref[...] + x_ref[...]
    o_ref[...] = jnp.maximum(y, 0.0)


def _params():
    return pltpu.CompilerParams(
        dimension_semantics=("arbitrary",),
        vmem_limit_bytes=64 * 1024 * 1024,
    )


def _conv1_call(x_l, wt, B):
    N, H, WC = x_l.shape
    C = wt.shape[1]
    G = N // B
    return pl.pallas_call(
        _conv1_kernel,
        out_shape=(
            jax.ShapeDtypeStruct((N, H, WC), jnp.bfloat16),
            jax.ShapeDtypeStruct((G, 1, WC), jnp.float32),
            jax.ShapeDtypeStruct((G, 1, WC), jnp.float32),
        ),
        grid=(G,),
        in_specs=[
            pl.BlockSpec((B, H, WC), lambda n: (n, 0, 0)),
            pl.BlockSpec((9, C, WC), lambda n: (0, 0, 0)),
        ],
        out_specs=(
            pl.BlockSpec((B, H, WC), lambda n: (n, 0, 0)),
            pl.BlockSpec((1, 1, WC), lambda n: (n, 0, 0)),
            pl.BlockSpec((1, 1, WC), lambda n: (n, 0, 0)),
        ),
        scratch_shapes=[pltpu.VMEM((3 * WC, WC), jnp.bfloat16)],
        compiler_params=_params(),
    )(x_l, wt)


def _conv2_stats_call(c1, scale, shift, wt, B):
    N, H, WC = c1.shape
    C = wt.shape[1]
    G = N // B
    return pl.pallas_call(
        _bn_relu_conv2_stats_kernel,
        out_shape=(
            jax.ShapeDtypeStruct((G, 1, WC), jnp.float32),
            jax.ShapeDtypeStruct((G, 1, WC), jnp.float32),
        ),
        grid=(G,),
        in_specs=[
            pl.BlockSpec((B, H, WC), lambda n: (n, 0, 0)),
            pl.BlockSpec((1, 1, WC), lambda n: (0, 0, 0)),
            pl.BlockSpec((1, 1, WC), lambda n: (0, 0, 0)),
            pl.BlockSpec((9, C, WC), lambda n: (0, 0, 0)),
        ],
        out_specs=(
            pl.BlockSpec((1, 1, WC), lambda n: (n, 0, 0)),
            pl.BlockSpec((1, 1, WC), lambda n: (n, 0, 0)),
        ),
        scratch_shapes=[pltpu.VMEM((3 * WC, WC), jnp.bfloat16)],
        compiler_params=_params(),
    )(c1, scale, shift, wt)


def _finish_call(c1, x_l, scale1, shift1, scale2, shift2, wt, B):
    N, H, WC = c1.shape
    C = wt.shape[1]
    G = N // B
    return pl.pallas_call(
        _conv2_finish_kernel,
        out_shape=jax.ShapeDtypeStruct((N, H, WC), jnp.float32),
        grid=(G,),
        in_specs=[
            pl.BlockSpec((B, H, WC), lambda n: (n, 0, 0)),
            pl.BlockSpec((B, H, WC), lambda n: (n, 0, 0)),
            pl.BlockSpec((1, 1, WC), lambda n: (0, 0, 0)),
            pl.BlockSpec((1, 1, WC), lambda n: (0, 0, 0)),
            pl.BlockSpec((1, 1, WC), lambda n: (0, 0, 0)),
            pl.BlockSpec((1, 1, WC), lambda n: (0, 0, 0)),
            pl.BlockSpec((9, C, WC), lambda n: (0, 0, 0)),
        ],
        out_specs=pl.BlockSpec((B, H, WC), lambda n: (n, 0, 0)),
        scratch_shapes=[pltpu.VMEM((3 * WC, WC), jnp.bfloat16)],
        compiler_params=_params(),
    )(c1, x_l, scale1, shift1, scale2, shift2, wt)


def _bn_affine(s_b, s2_b, gamma, beta, total, W, C):
    G = s_b.shape[0]
    s = jnp.sum(s_b.reshape(G * W, C), axis=0)
    s2 = jnp.sum(s2_b.reshape(G * W, C), axis=0)
    mean = s / total
    var = s2 / total - mean * mean
    scale = gamma * jax.lax.rsqrt(var + _EPS)
    shift = beta - mean * scale
    return jnp.tile(scale, W)[None, None], jnp.tile(shift, W)[None, None]


def _pick_block(n, targets=(128, 64, 32, 16, 8, 4, 2)):
    for t in targets:
        if n % t == 0:
            return t
    return 1


@jax.jit
def _residual_block_opt(x, w1, g1, b1, w2, g2, b2):
    N, H, W, C = x.shape
    WC = W * C
    B = _pick_block(N)

    wt1 = _tile_w(w1, W)
    wt2 = _tile_w(w2, W)
    x_l = x.reshape(N, H, WC)

    c1, s1, q1 = _conv1_call(x_l, wt1, B)
    scale1, shift1 = _bn_affine(s1, q1, g1, b1, N * H * W, W, C)

    s2, q2 = _conv2_stats_call(c1, scale1, shift1, wt2, B)
    scale2, shift2 = _bn_affine(s2, q2, g2, b2, N * H * W, W, C)

    out_l = _finish_call(c1, x_l, scale1, shift1, scale2, shift2, wt2, B)
    return out_l.reshape(N, H, W, C)


def kernel(x, w1, g1, b1, w2, g2, b2):
    return _residual_block_opt(x, w1, g1, b1, w2, g2, b2)


# speedup vs baseline: 1.0563x; 1.0563x over previous
import jax
import jax.numpy as jnp
from jax.experimental import pallas as pl
from jax.experimental.pallas import tpu as pltpu

_EPS = 1e-5


def _build_band(wt_ref, wb_scratch, W, C):
    WC = W * C
    u = jax.lax.broadcasted_iota(jnp.int32, (WC, WC), 0) // C
    v = jax.lax.broadcasted_iota(jnp.int32, (WC, WC), 1) // C
    d = v - u
    for dy in range(3):
        sec = jnp.zeros((WC, WC), jnp.float32)
        for dx in range(3):
            blk = jnp.broadcast_to(wt_ref[3 * dy + dx][None], (W, C, WC))
            sec = jnp.where(d == 1 - dx, blk.reshape(WC, WC), sec)
        wb_scratch[dy * WC:(dy + 1) * WC, :] = sec.astype(jnp.bfloat16)


def _tile_w(w, W):
    C = w.shape[-1]
    return jnp.tile(w.reshape(9, C, C), (1, 1, W))


def _conv_rows(xs, wb_ref, H):
    M, WC = xs.shape
    row = jax.lax.broadcasted_iota(jnp.int32, (M, 1), 0)
    x_up = jnp.where(row % H != 0, pltpu.roll(xs, 1, axis=0), 0.0)
    x_dn = jnp.where(row % H != (H - 1), pltpu.roll(xs, M - 1, axis=0), 0.0)
    xcat = jnp.concatenate([x_up, xs, x_dn], axis=1).astype(jnp.bfloat16)
    return jnp.dot(xcat, wb_ref[...], preferred_element_type=jnp.float32)


def _emit(acc, B, H, WC, o_ref, s_ref, s2_ref):
    o_ref[...] = acc.reshape(B, H, WC).astype(o_ref.dtype)
    s_ref[...] = jnp.sum(acc, axis=0, keepdims=True)[None]
    s2_ref[...] = jnp.sum(acc * acc, axis=0, keepdims=True)[None]


def _conv1_kernel(x_ref, wt_ref, o_ref, mean_ref, m2_ref, wb_scratch):
    B, H, WC = x_ref.shape
    C = wt_ref.shape[1]

    @pl.when(pl.program_id(0) == 0)
    def _():
        _build_band(wt_ref, wb_scratch, WC // C, C)

    acc = _conv_rows(x_ref[...].reshape(B * H, WC), wb_scratch, H)
    _emit(acc, B, H, WC, o_ref, mean_ref, m2_ref)


def _bn_relu_conv2_kernel(c_ref, scale_ref, shift_ref, wt_ref,
                          o_ref, mean_ref, m2_ref, wb_scratch):
    B, H, WC = c_ref.shape
    C = wt_ref.shape[1]

    @pl.when(pl.program_id(0) == 0)
    def _():
        _build_band(wt_ref, wb_scratch, WC // C, C)

    c = c_ref[...].astype(jnp.float32)
    h = jnp.maximum(c * scale_ref[...] + shift_ref[...], 0.0)
    acc = _conv_rows(h.reshape(B * H, WC), wb_scratch, H)
    _emit(acc, B, H, WC, o_ref, mean_ref, m2_ref)


def _bn_add_relu_kernel(c_ref, x_ref, scale_ref, shift_ref, o_ref):
    c = c_ref[...].astype(jnp.float32)
    o_ref[...] = jnp.maximum(
        c * scale_ref[...] + shift_ref[...] + x_ref[...], 0.0)


def _params():
    return pltpu.CompilerParams(
        dimension_semantics=("arbitrary",),
        vmem_limit_bytes=64 * 1024 * 1024,
    )


def _conv1_call(x_l, wt, B):
    N, H, WC = x_l.shape
    C = wt.shape[1]
    G = N // B
    return pl.pallas_call(
        _conv1_kernel,
        out_shape=(
            jax.ShapeDtypeStruct((N, H, WC), jnp.bfloat16),
            jax.ShapeDtypeStruct((G, 1, WC), jnp.float32),
            jax.ShapeDtypeStruct((G, 1, WC), jnp.float32),
        ),
        grid=(G,),
        in_specs=[
            pl.BlockSpec((B, H, WC), lambda n: (n, 0, 0)),
            pl.BlockSpec((9, C, WC), lambda n: (0, 0, 0)),
        ],
        out_specs=(
            pl.BlockSpec((B, H, WC), lambda n: (n, 0, 0)),
            pl.BlockSpec((1, 1, WC), lambda n: (n, 0, 0)),
            pl.BlockSpec((1, 1, WC), lambda n: (n, 0, 0)),
        ),
        scratch_shapes=[pltpu.VMEM((3 * WC, WC), jnp.bfloat16)],
        compiler_params=_params(),
    )(x_l, wt)


def _conv2_call(c1, scale, shift, wt, B):
    N, H, WC = c1.shape
    C = wt.shape[1]
    G = N // B
    return pl.pallas_call(
        _bn_relu_conv2_kernel,
        out_shape=(
            jax.ShapeDtypeStruct((N, H, WC), jnp.bfloat16),
            jax.ShapeDtypeStruct((G, 1, WC), jnp.float32),
            jax.ShapeDtypeStruct((G, 1, WC), jnp.float32),
        ),
        grid=(G,),
        in_specs=[
            pl.BlockSpec((B, H, WC), lambda n: (n, 0, 0)),
            pl.BlockSpec((1, 1, WC), lambda n: (0, 0, 0)),
            pl.BlockSpec((1, 1, WC), lambda n: (0, 0, 0)),
            pl.BlockSpec((9, C, WC), lambda n: (0, 0, 0)),
        ],
        out_specs=(
            pl.BlockSpec((B, H, WC), lambda n: (n, 0, 0)),
            pl.BlockSpec((1, 1, WC), lambda n: (n, 0, 0)),
            pl.BlockSpec((1, 1, WC), lambda n: (n, 0, 0)),
        ),
        scratch_shapes=[pltpu.VMEM((3 * WC, WC), jnp.bfloat16)],
        compiler_params=_params(),
    )(c1, scale, shift, wt)


def _finish_call(c2, x_l, scale, shift, B):
    N, H, WC = c2.shape
    G = N // B
    return pl.pallas_call(
        _bn_add_relu_kernel,
        out_shape=jax.ShapeDtypeStruct((N, H, WC), jnp.float32),
        grid=(G,),
        in_specs=[
            pl.BlockSpec((B, H, WC), lambda n: (n, 0, 0)),
            pl.BlockSpec((B, H, WC), lambda n: (n, 0, 0)),
            pl.BlockSpec((1, 1, WC), lambda n: (0, 0, 0)),
            pl.BlockSpec((1, 1, WC), lambda n: (0, 0, 0)),
        ],
        out_specs=pl.BlockSpec((B, H, WC), lambda n: (n, 0, 0)),
        compiler_params=_params(),
    )(c2, x_l, scale, shift)


def _bn_affine(s_b, s2_b, gamma, beta, total, W, C):
    G = s_b.shape[0]
    s = jnp.sum(s_b.reshape(G * W, C), axis=0)
    s2 = jnp.sum(s2_b.reshape(G * W, C), axis=0)
    mean = s / total
    var = s2 / total - mean * mean
    scale = gamma * jax.lax.rsqrt(var + _EPS)
    shift = beta - mean * scale
    return jnp.tile(scale, W)[None, None], jnp.tile(shift, W)[None, None]


def _pick_block(n, targets=(32, 16, 8, 4, 2)):
    for t in targets:
        if n % t == 0:
            return t
    return 1


@jax.jit
def _residual_block_opt(x, w1, g1, b1, w2, g2, b2):
    N, H, W, C = x.shape
    WC = W * C
    B = _pick_block(N, (128, 64, 32, 16, 8, 4, 2))
    B3 = _pick_block(N, (128, 64, 32, 16, 8, 4, 2))

    wt1 = _tile_w(w1, W)
    wt2 = _tile_w(w2, W)
    x_l = x.reshape(N, H, WC)
    c1, m1, q1 = _conv1_call(x_l, wt1, B)
    scale1, shift1 = _bn_affine(m1, q1, g1, b1, N * H * W, W, C)

    c2, m2, q2 = _conv2_call(c1, scale1, shift1, wt2, B)
    scale2, shift2 = _bn_affine(m2, q2, g2, b2, N * H * W, W, C)

    out_l = _finish_call(c2, x_l, scale2, shift2, B3)
    return out_l.reshape(N, H, W, C)


def kernel(x, w1, g1, b1, w2, g2, b2):
    return _residual_block_opt(x, w1, g1, b1, w2, g2, b2)
